# baseline (speedup 1.0000x reference)
import functools

import jax
import jax.numpy as jnp
from jax import lax
from jax.experimental import pallas as pl
from jax.experimental.pallas import tpu as pltpu
from jax.experimental.pallas import tpu_sc as plsc

VOCAB = 100000
D = 1024
B = 4
S = 4096
T = B * S

NC = 2
NS = 16
NW = NC * NS
TPW = T // NW
C = 8
NBUF = 4
NCH = TPW // C
LANES = 16


def _body(ids_hbm, pos_hbm, wte_hbm, wpe_hbm, out_hbm,
          idx_tok, idx_pos,
          ra0, rb0, ro0, ra1, rb1, ro1, ra2, rb2, ro2, ra3, rb3, ro3,
          sg0, sg1, sg2, sg3, so0, so1, so2, so3):
    wid = lax.axis_index("s") * NC + lax.axis_index("c")
    base = wid * TPW
    pltpu.sync_copy(ids_hbm.at[pl.ds(base, TPW)], idx_tok)
    pltpu.sync_copy(pos_hbm.at[pl.ds(base, TPW)], idx_pos)

    RA = (ra0, ra1, ra2, ra3)
    RB = (rb0, rb1, rb2, rb3)
    RO = (ro0, ro1, ro2, ro3)
    SG = (sg0, sg1, sg2, sg3)
    SO = (so0, so1, so2, so3)

    def issue_gathers(ch, b):
        c0 = ch * C
        pltpu.async_copy(wte_hbm.at[idx_tok.at[pl.ds(c0, C)]], RA[b], SG[b])
        pltpu.async_copy(wpe_hbm.at[idx_pos.at[pl.ds(c0, C)]], RB[b], SG[b])

    for b in range(NBUF):
        issue_gathers(b, b)

    NI = NCH // NBUF

    def it(i, carry):
        for b in range(NBUF):
            ch = i * NBUF + b
            pltpu.make_async_copy(wte_hbm.at[pl.ds(0, C)], RA[b], SG[b]).wait()
            pltpu.make_async_copy(wte_hbm.at[pl.ds(0, C)], RB[b], SG[b]).wait()

            @pl.when(i > 0)
            def _wait_out(_b=b):
                pltpu.make_async_copy(
                    RO[_b], out_hbm.at[pl.ds(0, C)], SO[_b]).wait()

            def row_body(rr, c2, _b=b):
                r = lax.shift_right_logical(rr, 1)
                h = lax.bitwise_and(rr, 1) * (D // 2)
                for j in range(D // LANES // 2):
                    sl = pl.ds(h + j * LANES, LANES)
                    RO[_b][r, sl] = RA[_b][r, sl] + RB[_b][r, sl]
                return c2
            lax.fori_loop(0, 2 * C, row_body, 0)

            @pl.when(i < NI - 1)
            def _prefetch(_ch=ch, _b=b):
                issue_gathers(_ch + NBUF, _b)

            pltpu.async_copy(RO[b], out_hbm.at[pl.ds(base + ch * C, C)], SO[b])
        return carry

    lax.fori_loop(0, NI, it, 0)
    for b in range(NBUF):
        pltpu.make_async_copy(RO[b], out_hbm.at[pl.ds(0, C)], SO[b]).wait()


_embed_call = functools.partial(
    pl.kernel,
    out_type=jax.ShapeDtypeStruct((T, D), jnp.float32),
    mesh=plsc.VectorSubcoreMesh(core_axis_name="c", subcore_axis_name="s"),
    scratch_types=(
        [pltpu.VMEM((TPW,), jnp.int32)] * 2
        + [pltpu.VMEM((C, D), jnp.float32)] * (3 * NBUF)
        + [pltpu.SemaphoreType.DMA] * (2 * NBUF)
    ),
)(_body)


def kernel(input_ids, position_ids, wte, wpe):
    ids = input_ids.reshape(T).astype(jnp.int32)
    pos = position_ids.reshape(T).astype(jnp.int32)
    out = _embed_call(ids, pos, wte, wpe)
    return out.reshape(B, S, D)


# speedup vs baseline: 1.5421x; 1.5421x over previous
import functools

import jax
import jax.numpy as jnp
from jax import lax
from jax.experimental import pallas as pl
from jax.experimental.pallas import tpu as pltpu
from jax.experimental.pallas import tpu_sc as plsc

VOCAB = 100000
D = 1024
B = 4
S = 4096
T = B * S

NC = 2
NS = 16
NW = NC * NS
TPW = T // NW
C = 8
NBUF = 4
NCH = TPW // C
LANES = 16


def _body(ids_hbm, pos_hbm, wte_hbm, wpe_hbm, out_hbm,
          idx_tok, idx_pos,
          ra0, rb0, ro0, ra1, rb1, ro1, ra2, rb2, ro2, ra3, rb3, ro3,
          sg0, sg1, sg2, sg3, so0, so1, so2, so3):
    wid = lax.axis_index("s") * NC + lax.axis_index("c")
    base = wid * TPW
    pltpu.sync_copy(ids_hbm.at[pl.ds(base, TPW)], idx_tok)
    pltpu.sync_copy(pos_hbm.at[pl.ds(base, TPW)], idx_pos)

    RA = (ra0, ra1, ra2, ra3)
    RB = (rb0, rb1, rb2, rb3)
    RO = (ro0, ro1, ro2, ro3)
    SG = (sg0, sg1, sg2, sg3)
    SO = (so0, so1, so2, so3)

    def issue_gathers(ch, b):
        c0 = ch * C
        pltpu.async_copy(wte_hbm.at[idx_tok.at[pl.ds(c0, C)]], RA[b], SG[b])
        pltpu.async_copy(wpe_hbm.at[idx_pos.at[pl.ds(c0, C)]], RB[b], SG[b])

    for b in range(NBUF):
        issue_gathers(b, b)

    NI = NCH // NBUF

    def it(i, carry):
        for b in range(NBUF):
            ch = i * NBUF + b
            pltpu.make_async_copy(wte_hbm.at[pl.ds(0, C)], RA[b], SG[b]).wait()
            pltpu.make_async_copy(wte_hbm.at[pl.ds(0, C)], RB[b], SG[b]).wait()

            @pl.when(i > 0)
            def _wait_out(_b=b):
                pltpu.make_async_copy(
                    RO[_b], out_hbm.at[pl.ds(0, C)], SO[_b]).wait()

            @plsc.parallel_loop(0, C, step=1, unroll=1)
            def _add_rows(r, _b=b):
                for j in range(D // LANES):
                    sl = pl.ds(j * LANES, LANES)
                    RO[_b][r, sl] = RA[_b][r, sl] + RB[_b][r, sl]

            @pl.when(i < NI - 1)
            def _prefetch(_ch=ch, _b=b):
                issue_gathers(_ch + NBUF, _b)

            pltpu.async_copy(RO[b], out_hbm.at[pl.ds(base + ch * C, C)], SO[b])
        return carry

    lax.fori_loop(0, NI, it, 0)
    for b in range(NBUF):
        pltpu.make_async_copy(RO[b], out_hbm.at[pl.ds(0, C)], SO[b]).wait()


_embed_call = functools.partial(
    pl.kernel,
    out_type=jax.ShapeDtypeStruct((T, D), jnp.float32),
    mesh=plsc.VectorSubcoreMesh(core_axis_name="c", subcore_axis_name="s"),
    scratch_types=(
        [pltpu.VMEM((TPW,), jnp.int32)] * 2
        + [pltpu.VMEM((C, D), jnp.float32)] * (3 * NBUF)
        + [pltpu.SemaphoreType.DMA] * (2 * NBUF)
    ),
)(_body)


def kernel(input_ids, position_ids, wte, wpe):
    ids = input_ids.reshape(T).astype(jnp.int32)
    pos = position_ids.reshape(T).astype(jnp.int32)
    out = _embed_call(ids, pos, wte, wpe)
    return out.reshape(B, S, D)
